# baseline (speedup 1.0000x reference)
import functools

import jax
import jax.numpy as jnp
from jax import lax
from jax.experimental import pallas as pl
from jax.experimental.pallas import tpu as pltpu
from jax.experimental.pallas import tpu_sc as plsc

_N_VOCAB = 1000000
_N_EMBED = 64
_BATCH = 16384

_info = plsc.get_sparse_core_info()
_NC = _info.num_cores
_NS = _info.num_subcores
_NW = _NC * _NS
_B_PER_W = _BATCH // _NW
_CHUNK = 128
_NCHUNK = _B_PER_W // _CHUNK


def _gather_kernel(table_hbm, idx_hbm, out_hbm, idx_v, rows_v, sem):
    wid = lax.axis_index("s") * _NC + lax.axis_index("c")
    pltpu.sync_copy(idx_hbm.at[wid], idx_v)
    copies = []
    for j in range(_NCHUNK):
        copies.append(
            pltpu.async_copy(table_hbm.at[idx_v.at[j]], rows_v.at[j], sem)
        )
    for c in copies:
        c.wait()
    pltpu.sync_copy(rows_v, out_hbm.at[wid])


@jax.jit
def kernel(input_words, in_embed_weight):
    idx = input_words.reshape(_NW, _NCHUNK, _CHUNK)
    mesh = plsc.VectorSubcoreMesh(core_axis_name="c", subcore_axis_name="s")
    out = pl.kernel(
        _gather_kernel,
        mesh=mesh,
        out_type=jax.ShapeDtypeStruct(
            (_NW, _NCHUNK, _CHUNK, _N_EMBED), jnp.float32
        ),
        scratch_types=[
            pltpu.VMEM((_NCHUNK, _CHUNK), jnp.int32),
            pltpu.VMEM((_NCHUNK, _CHUNK, _N_EMBED), jnp.float32),
            pltpu.SemaphoreType.DMA,
        ],
        compiler_params=pltpu.CompilerParams(use_tc_tiling_on_sc=False),
    )(in_embed_weight, idx)
    return out.reshape(_BATCH, _N_EMBED)


# speedup vs baseline: 1.6303x; 1.6303x over previous
import functools

import jax
import jax.numpy as jnp
from jax import lax
from jax.experimental import pallas as pl
from jax.experimental.pallas import tpu as pltpu
from jax.experimental.pallas import tpu_sc as plsc

_N_VOCAB = 1000000
_N_EMBED = 64
_BATCH = 16384

_info = plsc.get_sparse_core_info()
_NC = _info.num_cores
_NS = _info.num_subcores
_NW = _NC * _NS
_B_PER_W = _BATCH // _NW
_K = 16
_NBATCH = _B_PER_W // _K


def _gather_kernel(tbl_hbm, idx_hbm, out_hbm, idx_v, rows_v, sem):
    wid = lax.axis_index("s") * _NC + lax.axis_index("c")
    base = wid * _B_PER_W
    pltpu.sync_copy(idx_hbm.at[wid], idx_v)

    def batch_body(b, _):
        vblk = idx_v[pl.ds(b * _K, _K)]
        copies = []
        for l in range(_K):
            i = vblk[l]
            copies.append(
                pltpu.async_copy(
                    tbl_hbm.at[pl.ds(i, 1), :],
                    rows_v.at[pl.ds(b * _K + l, 1), :],
                    sem,
                )
            )
        for c in copies:
            c.wait()
        return ()

    lax.fori_loop(0, _NBATCH, batch_body, (), unroll=False)
    pltpu.sync_copy(rows_v, out_hbm.at[pl.ds(base, _B_PER_W)])


@jax.jit
def kernel(input_words, in_embed_weight):
    idx = input_words.reshape(_NW, _B_PER_W)
    mesh = plsc.VectorSubcoreMesh(core_axis_name="c", subcore_axis_name="s")
    out = pl.kernel(
        _gather_kernel,
        mesh=mesh,
        out_type=jax.ShapeDtypeStruct((_BATCH, _N_EMBED), jnp.float32),
        scratch_types=[
            pltpu.VMEM((_B_PER_W,), jnp.int32),
            pltpu.VMEM((_B_PER_W, _N_EMBED), jnp.float32),
            pltpu.SemaphoreType.DMA,
        ],
    )(in_embed_weight, idx)
    return out


# speedup vs baseline: 1.7183x; 1.0540x over previous
import functools

import jax
import jax.numpy as jnp
from jax import lax
from jax.experimental import pallas as pl
from jax.experimental.pallas import tpu as pltpu
from jax.experimental.pallas import tpu_sc as plsc

_N_VOCAB = 1000000
_N_EMBED = 64
_BATCH = 16384

_info = plsc.get_sparse_core_info()
_NC = _info.num_cores
_NS = _info.num_subcores
_NW = _NC * _NS
_B_PER_W = _BATCH // _NW
_K = 16
_NBATCH = _B_PER_W // _K


def _gather_kernel(tbl_hbm, idx_hbm, out_hbm, idx_v, rows_v, sem):
    wid = lax.axis_index("s") * _NC + lax.axis_index("c")
    base = wid * _B_PER_W
    pltpu.sync_copy(idx_hbm.at[wid], idx_v)

    def batch_body(b, _):
        vblk = idx_v[pl.ds(b * _K, _K)]
        for l in range(_K):
            i = vblk[l]
            pltpu.async_copy(
                tbl_hbm.at[pl.ds(i, 1), :],
                rows_v.at[pl.ds(b * _K + l, 1), :],
                sem,
            )
        return ()

    lax.fori_loop(0, _NBATCH, batch_body, (), unroll=False)

    def drain_body(b, _):
        pltpu.make_async_copy(
            tbl_hbm.at[pl.ds(0, 1), :], rows_v.at[pl.ds(0, 1), :], sem
        ).wait()
        return ()

    lax.fori_loop(0, _B_PER_W, drain_body, (), unroll=False)
    pltpu.sync_copy(rows_v, out_hbm.at[pl.ds(base, _B_PER_W)])


@jax.jit
def kernel(input_words, in_embed_weight):
    idx = input_words.reshape(_NW, _B_PER_W)
    mesh = plsc.VectorSubcoreMesh(core_axis_name="c", subcore_axis_name="s")
    out = pl.kernel(
        _gather_kernel,
        mesh=mesh,
        out_type=jax.ShapeDtypeStruct((_BATCH, _N_EMBED), jnp.float32),
        scratch_types=[
            pltpu.VMEM((_B_PER_W,), jnp.int32),
            pltpu.VMEM((_B_PER_W, _N_EMBED), jnp.float32),
            pltpu.SemaphoreType.DMA,
        ],
    )(in_embed_weight, idx)
    return out
